# baseline (speedup 1.0000x reference)
import jax
import jax.numpy as jnp
from jax.experimental import pallas as pl

N_HALF = 25000
D = 256
BR = 1000
NB = N_HALF // BR


def _proj_kernel(ctx_ref, smp_ref, w_ref, b_ref, out_ref):
    i = pl.program_id(0)

    @pl.when(i < NB)
    def _():
        out_ref[...] = (
            jnp.dot(ctx_ref[...], w_ref[...], preferred_element_type=jnp.float32)
            + b_ref[...]
        )

    @pl.when(i >= NB)
    def _():
        out_ref[...] = (
            jnp.dot(smp_ref[...], w_ref[...], preferred_element_type=jnp.float32)
            + b_ref[...]
        )


def kernel(context, sample, W_proj, b_proj):
    b2d = b_proj.reshape(1, D)
    out = pl.pallas_call(
        _proj_kernel,
        grid=(2 * NB,),
        in_specs=[
            pl.BlockSpec((BR, D), lambda i: (jnp.minimum(i, NB - 1), 0)),
            pl.BlockSpec((BR, D), lambda i: (jnp.maximum(i - NB, 0), 0)),
            pl.BlockSpec((D, D), lambda i: (0, 0)),
            pl.BlockSpec((1, D), lambda i: (0, 0)),
        ],
        out_specs=pl.BlockSpec((BR, D), lambda i: (i, 0)),
        out_shape=jax.ShapeDtypeStruct((2 * N_HALF, D), jnp.float32),
    )(context, sample, W_proj, b2d)
    return out


# speedup vs baseline: 1.6320x; 1.6320x over previous
import jax
import jax.numpy as jnp
from jax.experimental import pallas as pl
from jax.experimental.pallas import tpu as pltpu

N_HALF = 25000
D = 256
BR = 5000
NB = N_HALF // BR


def _proj_kernel(ctx_ref, smp_ref, w_ref, b_ref, out_ref):
    i = pl.program_id(0)

    @pl.when(i < NB)
    def _():
        out_ref[...] = (
            jnp.dot(ctx_ref[...], w_ref[...], preferred_element_type=jnp.float32)
            + b_ref[...]
        )

    @pl.when(i >= NB)
    def _():
        out_ref[...] = (
            jnp.dot(smp_ref[...], w_ref[...], preferred_element_type=jnp.float32)
            + b_ref[...]
        )


def kernel(context, sample, W_proj, b_proj):
    b2d = b_proj.reshape(1, D)
    out = pl.pallas_call(
        _proj_kernel,
        grid=(2 * NB,),
        in_specs=[
            pl.BlockSpec((BR, D), lambda i: (jnp.minimum(i, NB - 1), 0)),
            pl.BlockSpec((BR, D), lambda i: (jnp.maximum(i - NB, 0), 0)),
            pl.BlockSpec((D, D), lambda i: (0, 0)),
            pl.BlockSpec((1, D), lambda i: (0, 0)),
        ],
        out_specs=pl.BlockSpec((BR, D), lambda i: (i, 0)),
        out_shape=jax.ShapeDtypeStruct((2 * N_HALF, D), jnp.float32),
        compiler_params=pltpu.CompilerParams(
            dimension_semantics=("arbitrary",),
        ),
    )(context, sample, W_proj, b2d)
    return out
